# baseline (speedup 1.0000x reference)
import functools

import jax
import jax.numpy as jnp
from jax import lax
from jax.experimental import pallas as pl
from jax.experimental.pallas import tpu as pltpu
from jax.experimental.pallas import tpu_sc as plsc

_NBUF = 4
_LANES = 16


def _routing_kernel(idx_hbm, w_hbm, coeff_hbm,
                    idx_v, w_v, local, *, N, E, K):
    wid = lax.axis_index("s") * 2 + lax.axis_index("c")
    n_workers = N // _LANES

    @pl.when(wid < n_workers)
    def _work():
        base = wid * _LANES
        for r in range(_LANES):
            for c in range(E // _LANES):
                local[r, pl.ds(c * _LANES, _LANES)] = jnp.zeros(
                    (_LANES,), jnp.float32)
        rows = jnp.arange(_LANES, dtype=jnp.int32)
        for k in range(K):
            pltpu.sync_copy(idx_hbm.at[k, pl.ds(base, _LANES)], idx_v)
            pltpu.sync_copy(w_hbm.at[k, pl.ds(base, _LANES)], w_v)
            plsc.addupdate_scatter(local, [rows, idx_v[...]], w_v[...])
        pltpu.sync_copy(local, coeff_hbm.at[pl.ds(base, _LANES), :])


def _moe_kernel(coeff_ref, x_ref, scale_ref, gu_hbm, dw_hbm, out_ref,
                gu_buf, dw_buf, xb_ref, sem, *, F, E):
    lookahead = _NBUF - 1

    def start_copy(e, slot):
        pltpu.make_async_copy(gu_hbm.at[e], gu_buf.at[slot], sem.at[slot, 0]).start()
        pltpu.make_async_copy(dw_hbm.at[e], dw_buf.at[slot], sem.at[slot, 1]).start()

    for j in range(lookahead):
        start_copy(j, j)

    out_ref[...] = jnp.zeros_like(out_ref)
    xb_ref[...] = x_ref[...].astype(jnp.bfloat16)

    def body(e, _):
        slot = jax.lax.rem(e, _NBUF)

        @pl.when(e + lookahead < E)
        def _prefetch():
            start_copy(e + lookahead, jax.lax.rem(e + lookahead, _NBUF))

        pltpu.make_async_copy(gu_hbm.at[e], gu_buf.at[slot], sem.at[slot, 0]).wait()
        pltpu.make_async_copy(dw_hbm.at[e], dw_buf.at[slot], sem.at[slot, 1]).wait()

        onehot = (lax.broadcasted_iota(jnp.int32, (1, E), 1) == e)
        coeff = jnp.sum(jnp.where(onehot, coeff_ref[...], 0.0), axis=1,
                        keepdims=True) * scale_ref[e]

        gu = gu_buf[slot].astype(jnp.bfloat16)
        h = jnp.dot(xb_ref[...], gu, preferred_element_type=jnp.float32)
        gate = h[:, :F]
        up = h[:, F:]
        act = 0.5 * gate * (1.0 + jax.lax.erf(gate * 0.7071067811865476)) * up
        y = jnp.dot(act.astype(jnp.bfloat16), dw_buf[slot].astype(jnp.bfloat16),
                    preferred_element_type=jnp.float32)
        out_ref[...] += coeff * y
        return 0

    jax.lax.fori_loop(0, E, body, 0)


def kernel(x, weights, expert_indices, gate_up, down, per_expert_scale):
    B, L, D = x.shape
    K = weights.shape[-1]
    E, _, F2 = gate_up.shape
    F = F2 // 2
    N = B * L

    x_flat = x.reshape(N, D)
    w_kn = weights.reshape(N, K).T
    idx_kn = expert_indices.reshape(N, K).T

    mesh = plsc.VectorSubcoreMesh(core_axis_name="c", subcore_axis_name="s")
    coeff = pl.kernel(
        functools.partial(_routing_kernel, N=N, E=E, K=K),
        out_type=jax.ShapeDtypeStruct((N, E), jnp.float32),
        mesh=mesh,
        compiler_params=pltpu.CompilerParams(needs_layout_passes=False),
        scratch_types=[
            pltpu.VMEM((_LANES,), jnp.int32),
            pltpu.VMEM((_LANES,), jnp.float32),
            pltpu.VMEM((_LANES, E), jnp.float32),
        ],
    )(idx_kn, w_kn)

    out = pl.pallas_call(
        functools.partial(_moe_kernel, F=F, E=E),
        in_specs=[
            pl.BlockSpec(memory_space=pltpu.MemorySpace.VMEM),
            pl.BlockSpec(memory_space=pltpu.MemorySpace.VMEM),
            pl.BlockSpec(memory_space=pltpu.MemorySpace.SMEM),
            pl.BlockSpec(memory_space=pl.ANY),
            pl.BlockSpec(memory_space=pl.ANY),
        ],
        out_specs=pl.BlockSpec(memory_space=pltpu.MemorySpace.VMEM),
        out_shape=jax.ShapeDtypeStruct((N, D), jnp.float32),
        scratch_shapes=[
            pltpu.VMEM((_NBUF, D, F2), jnp.float32),
            pltpu.VMEM((_NBUF, F, D), jnp.float32),
            pltpu.VMEM((N, D), jnp.bfloat16),
            pltpu.SemaphoreType.DMA((_NBUF, 2)),
        ],
    )(coeff, x_flat, per_expert_scale, gate_up, down)

    return out.reshape(B, L, D)


# speedup vs baseline: 1.1438x; 1.1438x over previous
import functools

import jax
import jax.numpy as jnp
from jax.experimental import pallas as pl
from jax.experimental.pallas import tpu as pltpu

_NBUF = 4
_EPB = 2


def _moe_kernel(idx_ref, w_ref, x_ref, scale_ref, gu_hbm, dw_hbm, out_ref,
                gu_buf, dw_buf, xb_ref, sem, *, F, E):
    lookahead = _NBUF - 1

    def start_copy(g, slot):
        pltpu.make_async_copy(gu_hbm.at[pl.ds(g * _EPB, _EPB)], gu_buf.at[slot],
                              sem.at[slot, 0]).start()
        pltpu.make_async_copy(dw_hbm.at[pl.ds(g * _EPB, _EPB)], dw_buf.at[slot],
                              sem.at[slot, 1]).start()

    for j in range(lookahead):
        start_copy(j, j)

    out_ref[...] = jnp.zeros_like(out_ref)
    xb_ref[...] = x_ref[...].astype(jnp.bfloat16)

    def body(g, _):
        slot = jax.lax.rem(g, _NBUF)

        @pl.when(g + lookahead < E // _EPB)
        def _prefetch():
            start_copy(g + lookahead, jax.lax.rem(g + lookahead, _NBUF))

        pltpu.make_async_copy(gu_hbm.at[pl.ds(g * _EPB, _EPB)], gu_buf.at[slot],
                              sem.at[slot, 0]).wait()
        pltpu.make_async_copy(dw_hbm.at[pl.ds(g * _EPB, _EPB)], dw_buf.at[slot],
                              sem.at[slot, 1]).wait()

        for sub in range(_EPB):
            e = g * _EPB + sub
            mask = idx_ref[...] == e
            coeff = jnp.sum(jnp.where(mask, w_ref[...], 0.0), axis=1,
                            keepdims=True)
            coeff = coeff * scale_ref[e]

            gu = gu_buf[slot, sub].astype(jnp.bfloat16)
            h = jnp.dot(xb_ref[...], gu, preferred_element_type=jnp.float32)
            gate = h[:, :F]
            up = h[:, F:]
            act = 0.5 * gate * (1.0 + jax.lax.erf(gate * 0.7071067811865476)) * up
            y = jnp.dot(act.astype(jnp.bfloat16),
                        dw_buf[slot, sub].astype(jnp.bfloat16),
                        preferred_element_type=jnp.float32)
            out_ref[...] += coeff * y
        return 0

    jax.lax.fori_loop(0, E // _EPB, body, 0)


def kernel(x, weights, expert_indices, gate_up, down, per_expert_scale):
    B, L, D = x.shape
    K = weights.shape[-1]
    E, _, F2 = gate_up.shape
    F = F2 // 2
    N = B * L

    x_flat = x.reshape(N, D)
    w_flat = weights.reshape(N, K)
    idx_flat = expert_indices.reshape(N, K)

    out = pl.pallas_call(
        functools.partial(_moe_kernel, F=F, E=E),
        in_specs=[
            pl.BlockSpec(memory_space=pltpu.MemorySpace.VMEM),
            pl.BlockSpec(memory_space=pltpu.MemorySpace.VMEM),
            pl.BlockSpec(memory_space=pltpu.MemorySpace.VMEM),
            pl.BlockSpec(memory_space=pltpu.MemorySpace.SMEM),
            pl.BlockSpec(memory_space=pl.ANY),
            pl.BlockSpec(memory_space=pl.ANY),
        ],
        out_specs=pl.BlockSpec(memory_space=pltpu.MemorySpace.VMEM),
        out_shape=jax.ShapeDtypeStruct((N, D), jnp.float32),
        scratch_shapes=[
            pltpu.VMEM((_NBUF, _EPB, D, F2), jnp.float32),
            pltpu.VMEM((_NBUF, _EPB, F, D), jnp.float32),
            pltpu.VMEM((N, D), jnp.bfloat16),
            pltpu.SemaphoreType.DMA((_NBUF, 2)),
        ],
    )(idx_flat, w_flat, x_flat, per_expert_scale, gate_up, down)

    return out.reshape(B, L, D)


# speedup vs baseline: 1.2114x; 1.0591x over previous
import functools

import jax
import jax.numpy as jnp
from jax.experimental import pallas as pl
from jax.experimental.pallas import tpu as pltpu

_NBUF = 3


def _moe_kernel(idx_ref, w_ref, x_ref, scale_ref, gu_hbm, dw_hbm, out_ref,
                gu_buf, dw_buf, xb_ref, sem, *, F, E):
    lookahead = _NBUF - 1

    def start_copy(e, slot):
        pltpu.make_async_copy(gu_hbm.at[e], gu_buf.at[slot], sem.at[slot, 0]).start()
        pltpu.make_async_copy(dw_hbm.at[e], dw_buf.at[slot], sem.at[slot, 1]).start()

    for j in range(lookahead):
        start_copy(j, j)

    out_ref[...] = jnp.zeros_like(out_ref)
    xb_ref[...] = x_ref[...].astype(jnp.bfloat16)

    def body(e, _):
        slot = jax.lax.rem(e, _NBUF)

        @pl.when(e + lookahead < E)
        def _prefetch():
            start_copy(e + lookahead, jax.lax.rem(e + lookahead, _NBUF))

        pltpu.make_async_copy(gu_hbm.at[e], gu_buf.at[slot], sem.at[slot, 0]).wait()
        pltpu.make_async_copy(dw_hbm.at[e], dw_buf.at[slot], sem.at[slot, 1]).wait()

        mask = idx_ref[...] == e
        coeff = jnp.sum(jnp.where(mask, w_ref[...], 0.0), axis=1, keepdims=True)
        coeff = coeff * scale_ref[e]

        gu = gu_buf[slot].astype(jnp.bfloat16)
        h = jnp.dot(xb_ref[...], gu, preferred_element_type=jnp.float32)
        gate = h[:, :F]
        up = h[:, F:]
        act = 0.5 * gate * (1.0 + jax.lax.erf(gate * 0.7071067811865476)) * up
        y = jnp.dot(act.astype(jnp.bfloat16), dw_buf[slot].astype(jnp.bfloat16),
                    preferred_element_type=jnp.float32)
        out_ref[...] += coeff * y
        return 0

    jax.lax.fori_loop(0, E, body, 0)


def kernel(x, weights, expert_indices, gate_up, down, per_expert_scale):
    B, L, D = x.shape
    K = weights.shape[-1]
    E, _, F2 = gate_up.shape
    F = F2 // 2
    N = B * L

    x_flat = x.reshape(N, D)
    w_flat = weights.reshape(N, K)
    idx_flat = expert_indices.reshape(N, K)

    out = pl.pallas_call(
        functools.partial(_moe_kernel, F=F, E=E),
        in_specs=[
            pl.BlockSpec(memory_space=pltpu.MemorySpace.VMEM),
            pl.BlockSpec(memory_space=pltpu.MemorySpace.VMEM),
            pl.BlockSpec(memory_space=pltpu.MemorySpace.VMEM),
            pl.BlockSpec(memory_space=pltpu.MemorySpace.SMEM),
            pl.BlockSpec(memory_space=pl.ANY),
            pl.BlockSpec(memory_space=pl.ANY),
        ],
        out_specs=pl.BlockSpec(memory_space=pltpu.MemorySpace.VMEM),
        out_shape=jax.ShapeDtypeStruct((N, D), jnp.float32),
        scratch_shapes=[
            pltpu.VMEM((_NBUF, D, F2), jnp.float32),
            pltpu.VMEM((_NBUF, F, D), jnp.float32),
            pltpu.VMEM((N, D), jnp.bfloat16),
            pltpu.SemaphoreType.DMA((_NBUF, 2)),
        ],
    )(idx_flat, w_flat, x_flat, per_expert_scale, gate_up, down)

    return out.reshape(B, L, D)
